# Initial kernel scaffold; baseline (speedup 1.0000x reference)
import jax
import jax.numpy as jnp
from jax.experimental import pallas as pl


def kernel(ctx_coords, ctx_values, tgt_coords, params):
    raise NotImplementedError("write your pallas kernel here")


# speedup vs baseline: 4.6808x; 4.6808x over previous
import functools

import jax
import jax.numpy as jnp
import numpy as np
from jax.experimental import pallas as pl

_NUM_NBHD = 25
_COEFF = 0.3
_TG = 1024
_C_HI = np.float32(0.1)
_C_LO = np.float32(np.float64(0.1) - np.float64(_C_HI))
_D_HI = np.float32(0.2)
_D_LO = np.float32(np.float64(0.2) - np.float64(_D_HI))

_RT = 256
_RN = 256


def _swish(x):
    return x * jax.nn.sigmoid(x)


def _two_sum(a, b):
    s = a + b
    bb = s - a
    return s, (a - (s - bb)) + (b - bb)


def _feat_body(C, t_row_ref, ctx_ref, vals_ref, preW_ref, preb_ref, sc_ref,
               v0_ref):
    t_row = t_row_ref[:, :]
    ctx = jnp.reshape(ctx_ref[:, :, :], (1, C))
    vals = vals_ref[:, :]
    ls = sc_ref[0, 0]
    os_ = sc_ref[0, 1]
    dd = (t_row - ctx) / ls
    K = os_ * jnp.exp(-0.5 * dd * dd)
    h0 = jnp.sum(K, axis=1, keepdims=True)
    h1 = jnp.dot(K, vals, preferred_element_type=jnp.float32)
    tv = jnp.concatenate([t_row, h0, h1 / (h0 + 1e-8)], axis=1)
    z = jnp.dot(tv, preW_ref[:, :], preferred_element_type=jnp.float32)
    v0_ref[:, :] = jax.nn.sigmoid(z + preb_ref[:, :])


def _knn_body(N, t_row_ref, t_full_ref, pen_ref, idx_ref):
    t_row = t_row_ref[:, :]
    t_full = jnp.reshape(t_full_ref[:, :, :], (1, N))
    pen = jnp.reshape(pen_ref[:, :, :], (1, N))
    d = jnp.abs(t_row - t_full) + pen
    cols = jax.lax.broadcasted_iota(jnp.int32, d.shape, 1)
    picked = []
    for _ in range(_NUM_NBHD):
        m = jnp.min(d, axis=1, keepdims=True)
        i = jnp.min(jnp.where(d == m, cols, N), axis=1, keepdims=True)
        picked.append(i)
        d = jnp.where(cols == i, jnp.inf, d)
    idx_ref[:, :] = jnp.concatenate(picked, axis=1)


def _layer_body(N, cin, idx_ref, t_row_ref, vt_ref, w1_ref, b1_ref, w2_ref,
                b2_ref, w3_ref, b3_ref, wl_ref, bl_ref, out_ref):
    f32 = jnp.float32
    idx = idx_ref[:, :]
    t_row = t_row_ref[:, :]
    vt = vt_ref[:, :]
    cols = jax.lax.broadcasted_iota(jnp.int32, (_RN, N), 1)
    cf = cin * 16
    je = jax.lax.broadcasted_iota(jnp.int32, (cin, cf), 1)
    re = jax.lax.broadcasted_iota(jnp.int32, (cin, cf), 0)
    E1 = (je // 16 == re).astype(f32)
    jf = jax.lax.broadcasted_iota(jnp.int32, (16, cf), 1)
    rf = jax.lax.broadcasted_iota(jnp.int32, (16, cf), 0)
    E2 = (jf % 16 == rf).astype(f32)
    acc = jnp.zeros((_RN, cf), dtype=f32)
    for k in range(_NUM_NBHD):
        oh = (cols == idx[:, k:k + 1]).astype(f32)
        g = jnp.dot(oh, vt, preferred_element_type=f32)
        nv = g[:, :cin]
        nt = g[:, cin:cin + 1]
        a = t_row - nt
        h = _swish(a * w1_ref[:, :] + b1_ref[:, :])
        h = _swish(jnp.dot(h, w2_ref[:, :], preferred_element_type=f32)
                   + b2_ref[:, :])
        w = (jnp.dot(h, w3_ref[:, :], preferred_element_type=f32)
             + b3_ref[:, :]) * _COEFF
        acc = acc + jnp.dot(nv, E1, preferred_element_type=f32) * \
            jnp.dot(w, E2, preferred_element_type=f32)
    flat = acc / _NUM_NBHD
    z = jnp.dot(flat, wl_ref[:, :], preferred_element_type=f32)
    out_ref[:, :] = jax.nn.relu(z + bl_ref[:, :])


def _head_body(Tt, f_ref, mW_ref, mb_ref, vW_ref, vb_ref, mean_ref, var_ref):
    f = f_ref[:, :]
    mean_ref[:, :] = (jnp.dot(f, mW_ref[:, :], preferred_element_type=jnp.float32)
                      + mb_ref[:, :])
    vd = jax.nn.softplus(
        jnp.dot(f, vW_ref[:, :], preferred_element_type=jnp.float32)
        + vb_ref[:, :])
    r = jax.lax.broadcasted_iota(jnp.int32, (Tt, Tt), 0)
    c = jax.lax.broadcasted_iota(jnp.int32, (Tt, Tt), 1)
    var_ref[0, :, :] = jnp.where(r == c, jnp.maximum(vd, 1e-8),
                                 jnp.float32(1e-8))


def kernel(ctx_coords, ctx_values, tgt_coords, params):
    B, C, _ = ctx_coords.shape
    Tt = tgt_coords.shape[1]
    N = _TG + Tt
    f32 = jnp.float32

    tmp = jnp.concatenate([jnp.reshape(ctx_coords, (-1,)),
                           jnp.reshape(tgt_coords, (-1,))])
    mn = jnp.min(tmp)
    mx = jnp.max(tmp)
    s_lo, e_lo = _two_sum(mn, -_C_HI)
    lower = s_lo + (e_lo - _C_LO)
    s_hi, e_hi = _two_sum(mx, _C_HI)
    upper = s_hi + (e_hi + _C_LO)
    d_s, d_e = _two_sum(mx, -mn)
    p_s, p_e = _two_sum(d_s, _D_HI)
    q = (p_e + d_e) + _D_LO
    a = 64.0 * p_s
    fa = jnp.floor(a)
    num_t_f = fa + jnp.floor((a - fa) + 64.0 * q)
    num_t = jnp.maximum(num_t_f, 1.0).astype(jnp.int32)

    div = jnp.maximum(num_t - 1, 1).astype(f32)
    delta = (upper - lower) / div
    iota = jnp.arange(_TG, dtype=f32)
    t_grid = lower + iota * delta
    t = jnp.concatenate(
        [jnp.broadcast_to(t_grid[None, :, None], (B, _TG, 1)), tgt_coords],
        axis=1)
    t_flat = jnp.reshape(t, (B * N, 1))
    t_wide = jnp.reshape(t, (B, 1, N))

    valid = jnp.concatenate([jnp.arange(_TG) < num_t,
                             jnp.ones((Tt,), dtype=bool)])
    pen = jnp.where(valid, f32(0), f32(jnp.inf)).reshape(1, 1, N)

    ctx_wide = jnp.reshape(ctx_coords, (B, 1, C))
    vals_flat = jnp.reshape(ctx_values, (B * C, 1))
    sc = jnp.stack([jnp.exp(params["log_lengthscale"]),
                    jnp.exp(params["log_outputscale"])]).reshape(1, 2)

    nt_r = N // _RT
    full = lambda b, r: (b, 0)
    wide = lambda b, r: (b, 0, 0)
    cst = lambda b, r: (0, 0)
    cst3 = lambda b, r: (0, 0, 0)
    row_rt = lambda b, r: (b * nt_r + r, 0)

    v0 = pl.pallas_call(
        functools.partial(_feat_body, C),
        grid=(B, nt_r),
        in_specs=[
            pl.BlockSpec((_RT, 1), row_rt),
            pl.BlockSpec((1, 1, C), wide),
            pl.BlockSpec((C, 1), full),
            pl.BlockSpec((3, 8), cst),
            pl.BlockSpec((1, 8), cst),
            pl.BlockSpec((1, 2), cst),
        ],
        out_specs=pl.BlockSpec((_RT, 8), row_rt),
        out_shape=jax.ShapeDtypeStruct((B * N, 8), f32),
    )(t_flat, ctx_wide, vals_flat, params["pre_W"],
      params["pre_b"].reshape(1, 8), sc)

    idx = pl.pallas_call(
        functools.partial(_knn_body, N),
        grid=(B, nt_r),
        in_specs=[
            pl.BlockSpec((_RT, 1), row_rt),
            pl.BlockSpec((1, 1, N), wide),
            pl.BlockSpec((1, 1, N), cst3),
        ],
        out_specs=pl.BlockSpec((_RT, 25), row_rt),
        out_shape=jax.ShapeDtypeStruct((B * N, 25), jnp.int32),
    )(t_flat, t_wide, pen)

    v = v0
    nt_n = N // _RN
    row_rn = lambda b, r: (b * nt_n + r, 0)
    for lw in params["layers"]:
        cin = v.shape[1]
        cout = lw["wl"].shape[1]
        vt = jnp.concatenate([v, t_flat], axis=1)
        v = pl.pallas_call(
            functools.partial(_layer_body, N, cin),
            grid=(B, nt_n),
            in_specs=[
                pl.BlockSpec((_RN, 25), row_rn),
                pl.BlockSpec((_RN, 1), row_rn),
                pl.BlockSpec((N, cin + 1), full),
                pl.BlockSpec((1, 32), cst),
                pl.BlockSpec((1, 32), cst),
                pl.BlockSpec((32, 32), cst),
                pl.BlockSpec((1, 32), cst),
                pl.BlockSpec((32, 16), cst),
                pl.BlockSpec((1, 16), cst),
                pl.BlockSpec((cin * 16, cout), cst),
                pl.BlockSpec((1, cout), cst),
            ],
            out_specs=pl.BlockSpec((_RN, cout), row_rn),
            out_shape=jax.ShapeDtypeStruct((B * N, cout), f32),
        )(idx, t_flat, vt, lw["w1"], lw["b1"].reshape(1, 32), lw["w2"],
          lw["b2"].reshape(1, 32), lw["w3"], lw["b3"].reshape(1, 16),
          lw["wl"], lw["bl"].reshape(1, cout))

    mean2, var = pl.pallas_call(
        functools.partial(_head_body, Tt),
        grid=(B,),
        in_specs=[
            pl.BlockSpec((Tt, 8), lambda b: (2 * b + 1, 0)),
            pl.BlockSpec((8, 1), lambda b: (0, 0)),
            pl.BlockSpec((1, 1), lambda b: (0, 0)),
            pl.BlockSpec((8, 1), lambda b: (0, 0)),
            pl.BlockSpec((1, 1), lambda b: (0, 0)),
        ],
        out_specs=[
            pl.BlockSpec((Tt, 1), lambda b: (b, 0)),
            pl.BlockSpec((1, Tt, Tt), lambda b: (b, 0, 0)),
        ],
        out_shape=[
            jax.ShapeDtypeStruct((B * Tt, 1), f32),
            jax.ShapeDtypeStruct((B, Tt, Tt), f32),
        ],
    )(v, params["mean_W"], params["mean_b"].reshape(1, 1),
      params["var_W"], params["var_b"].reshape(1, 1))

    mean = jnp.reshape(mean2, (B, Tt))
    return (mean, var)


# speedup vs baseline: 7.5088x; 1.6042x over previous
import functools

import jax
import jax.numpy as jnp
import numpy as np
from jax import lax
from jax.experimental import pallas as pl
from jax.experimental.pallas import tpu as pltpu
from jax.experimental.pallas import tpu_sc as plsc

_SC_CORES = 2
_SC_SUBCORES = 16
_SC_WORKERS = _SC_CORES * _SC_SUBCORES

_NUM_NBHD = 25
_COEFF = 0.3
_TG = 1024
_C_HI = np.float32(0.1)
_C_LO = np.float32(np.float64(0.1) - np.float64(_C_HI))
_D_HI = np.float32(0.2)
_D_LO = np.float32(np.float64(0.2) - np.float64(_D_HI))

_RT = 256
_RN = 256


def _swish(x):
    return x * jax.nn.sigmoid(x)


def _two_sum(a, b):
    s = a + b
    bb = s - a
    return s, (a - (s - bb)) + (b - bb)


def _feat_body(C, t_row_ref, ctx_ref, vals_ref, preW_ref, preb_ref, sc_ref,
               v0_ref):
    t_row = t_row_ref[:, :]
    ctx = jnp.reshape(ctx_ref[:, :, :], (1, C))
    vals = vals_ref[:, :]
    ls = sc_ref[0, 0]
    os_ = sc_ref[0, 1]
    dd = (t_row - ctx) / ls
    K = os_ * jnp.exp(-0.5 * dd * dd)
    h0 = jnp.sum(K, axis=1, keepdims=True)
    h1 = jnp.dot(K, vals, preferred_element_type=jnp.float32)
    tv = jnp.concatenate([t_row, h0, h1 / (h0 + 1e-8)], axis=1)
    z = jnp.dot(tv, preW_ref[:, :], preferred_element_type=jnp.float32)
    v0_ref[:, :] = jax.nn.sigmoid(z + preb_ref[:, :])


def _knn_body(N, t_row_ref, t_full_ref, pen_ref, idx_ref):
    t_row = t_row_ref[:, :]
    t_full = jnp.reshape(t_full_ref[:, :, :], (1, N))
    pen = jnp.reshape(pen_ref[:, :, :], (1, N))
    d = jnp.abs(t_row - t_full) + pen
    cols = jax.lax.broadcasted_iota(jnp.int32, d.shape, 1)
    base = pl.program_id(0) * N
    picked = []
    for _ in range(_NUM_NBHD):
        m = jnp.min(d, axis=1, keepdims=True)
        i = jnp.min(jnp.where(d == m, cols, N), axis=1, keepdims=True)
        picked.append(i + base)
        d = jnp.where(cols == i, jnp.inf, d)
    idx_ref[:, :] = jnp.concatenate(picked, axis=1)


def _sc_gather(table, idx_flat, D):
    G = idx_flat.shape[0]
    b_per_w = G // _SC_WORKERS
    ch = 800
    n_ch = b_per_w // ch
    mesh = plsc.VectorSubcoreMesh(core_axis_name="c", subcore_axis_name="s")

    @functools.partial(
        pl.kernel, mesh=mesh,
        out_type=jax.ShapeDtypeStruct((G, D), jnp.float32),
        scratch_types=[
            pltpu.VMEM((ch,), jnp.int32),
            pltpu.VMEM((ch, D), jnp.float32),
            pltpu.SemaphoreType.DMA,
        ],
    )
    def k(table_hbm, idx_hbm, out_hbm, idx_v, rows_v, sem):
        wid = lax.axis_index("s") * _SC_CORES + lax.axis_index("c")
        base = wid * b_per_w
        for ci in range(n_ch):
            off = base + ci * ch
            pltpu.sync_copy(idx_hbm.at[pl.ds(off, ch)], idx_v)
            pltpu.async_copy(table_hbm.at[idx_v], rows_v, sem).wait()
            pltpu.sync_copy(rows_v, out_hbm.at[pl.ds(off, ch)])

    return k(table, idx_flat)


def _combine_body(cin, g_ref, t_row_ref, w1_ref, b1_ref, w2_ref, b2_ref,
                  w3_ref, b3_ref, wl_ref, bl_ref, out_ref):
    f32 = jnp.float32
    t_row = t_row_ref[:, :]
    cf = cin * 16
    je = jax.lax.broadcasted_iota(jnp.int32, (cin, cf), 1)
    re = jax.lax.broadcasted_iota(jnp.int32, (cin, cf), 0)
    E1 = (je // 16 == re).astype(f32)
    jf = jax.lax.broadcasted_iota(jnp.int32, (16, cf), 1)
    rf = jax.lax.broadcasted_iota(jnp.int32, (16, cf), 0)
    E2 = (jf % 16 == rf).astype(f32)
    acc = jnp.zeros((_RN, cf), dtype=f32)
    for k in range(_NUM_NBHD):
        gk = g_ref[k, :, :]
        nv = gk[:, :cin]
        nt = gk[:, cin:cin + 1]
        a = t_row - nt
        h = _swish(a * w1_ref[:, :] + b1_ref[:, :])
        h = _swish(jnp.dot(h, w2_ref[:, :], preferred_element_type=f32)
                   + b2_ref[:, :])
        w = (jnp.dot(h, w3_ref[:, :], preferred_element_type=f32)
             + b3_ref[:, :]) * _COEFF
        acc = acc + jnp.dot(nv, E1, preferred_element_type=f32) * \
            jnp.dot(w, E2, preferred_element_type=f32)
    flat = acc / _NUM_NBHD
    z = jnp.dot(flat, wl_ref[:, :], preferred_element_type=f32)
    out_ref[:, :] = jax.nn.relu(z + bl_ref[:, :])


def _head_body(Tt, f_ref, mW_ref, mb_ref, vW_ref, vb_ref, mean_ref, var_ref):
    f = f_ref[:, :]
    mean_ref[:, :] = (jnp.dot(f, mW_ref[:, :], preferred_element_type=jnp.float32)
                      + mb_ref[:, :])
    vd = jax.nn.softplus(
        jnp.dot(f, vW_ref[:, :], preferred_element_type=jnp.float32)
        + vb_ref[:, :])
    r = jax.lax.broadcasted_iota(jnp.int32, (Tt, Tt), 0)
    c = jax.lax.broadcasted_iota(jnp.int32, (Tt, Tt), 1)
    var_ref[0, :, :] = jnp.where(r == c, jnp.maximum(vd, 1e-8),
                                 jnp.float32(1e-8))


def kernel(ctx_coords, ctx_values, tgt_coords, params):
    B, C, _ = ctx_coords.shape
    Tt = tgt_coords.shape[1]
    N = _TG + Tt
    f32 = jnp.float32

    tmp = jnp.concatenate([jnp.reshape(ctx_coords, (-1,)),
                           jnp.reshape(tgt_coords, (-1,))])
    mn = jnp.min(tmp)
    mx = jnp.max(tmp)
    s_lo, e_lo = _two_sum(mn, -_C_HI)
    lower = s_lo + (e_lo - _C_LO)
    s_hi, e_hi = _two_sum(mx, _C_HI)
    upper = s_hi + (e_hi + _C_LO)
    d_s, d_e = _two_sum(mx, -mn)
    p_s, p_e = _two_sum(d_s, _D_HI)
    q = (p_e + d_e) + _D_LO
    a = 64.0 * p_s
    fa = jnp.floor(a)
    num_t_f = fa + jnp.floor((a - fa) + 64.0 * q)
    num_t = jnp.maximum(num_t_f, 1.0).astype(jnp.int32)

    div = jnp.maximum(num_t - 1, 1).astype(f32)
    delta = (upper - lower) / div
    iota = jnp.arange(_TG, dtype=f32)
    t_grid = lower + iota * delta
    t = jnp.concatenate(
        [jnp.broadcast_to(t_grid[None, :, None], (B, _TG, 1)), tgt_coords],
        axis=1)
    t_flat = jnp.reshape(t, (B * N, 1))
    t_wide = jnp.reshape(t, (B, 1, N))

    valid = jnp.concatenate([jnp.arange(_TG) < num_t,
                             jnp.ones((Tt,), dtype=bool)])
    pen = jnp.where(valid, f32(0), f32(jnp.inf)).reshape(1, 1, N)

    ctx_wide = jnp.reshape(ctx_coords, (B, 1, C))
    vals_flat = jnp.reshape(ctx_values, (B * C, 1))
    sc = jnp.stack([jnp.exp(params["log_lengthscale"]),
                    jnp.exp(params["log_outputscale"])]).reshape(1, 2)

    nt_r = N // _RT
    full = lambda b, r: (b, 0)
    wide = lambda b, r: (b, 0, 0)
    cst = lambda b, r: (0, 0)
    cst3 = lambda b, r: (0, 0, 0)
    row_rt = lambda b, r: (b * nt_r + r, 0)

    v0 = pl.pallas_call(
        functools.partial(_feat_body, C),
        grid=(B, nt_r),
        in_specs=[
            pl.BlockSpec((_RT, 1), row_rt),
            pl.BlockSpec((1, 1, C), wide),
            pl.BlockSpec((C, 1), full),
            pl.BlockSpec((3, 8), cst),
            pl.BlockSpec((1, 8), cst),
            pl.BlockSpec((1, 2), cst),
        ],
        out_specs=pl.BlockSpec((_RT, 8), row_rt),
        out_shape=jax.ShapeDtypeStruct((B * N, 8), f32),
    )(t_flat, ctx_wide, vals_flat, params["pre_W"],
      params["pre_b"].reshape(1, 8), sc)

    idx = pl.pallas_call(
        functools.partial(_knn_body, N),
        grid=(B, nt_r),
        in_specs=[
            pl.BlockSpec((_RT, 1), row_rt),
            pl.BlockSpec((1, 1, N), wide),
            pl.BlockSpec((1, 1, N), cst3),
        ],
        out_specs=pl.BlockSpec((_RT, 25), row_rt),
        out_shape=jax.ShapeDtypeStruct((B * N, 25), jnp.int32),
    )(t_flat, t_wide, pen)

    gidx_flat = jnp.reshape(jnp.swapaxes(idx, 0, 1), (-1,))

    v = v0
    M = B * N
    nblk = M // _RN
    row1 = lambda i: (i, 0)
    cst1 = lambda i: (0, 0)
    for lw in params["layers"]:
        cin = v.shape[1]
        cout = lw["wl"].shape[1]
        D = 128
        vt = jnp.concatenate(
            [v, t_flat, jnp.zeros((M, D - cin - 1), f32)], axis=1)
        g = _sc_gather(vt, gidx_flat, D)
        g3 = jnp.reshape(g, (_NUM_NBHD, M, D))
        v = pl.pallas_call(
            functools.partial(_combine_body, cin),
            grid=(nblk,),
            in_specs=[
                pl.BlockSpec((_NUM_NBHD, _RN, D), lambda i: (0, i, 0)),
                pl.BlockSpec((_RN, 1), row1),
                pl.BlockSpec((1, 32), cst1),
                pl.BlockSpec((1, 32), cst1),
                pl.BlockSpec((32, 32), cst1),
                pl.BlockSpec((1, 32), cst1),
                pl.BlockSpec((32, 16), cst1),
                pl.BlockSpec((1, 16), cst1),
                pl.BlockSpec((cin * 16, cout), cst1),
                pl.BlockSpec((1, cout), cst1),
            ],
            out_specs=pl.BlockSpec((_RN, cout), row1),
            out_shape=jax.ShapeDtypeStruct((M, cout), f32),
        )(g3, t_flat, lw["w1"], lw["b1"].reshape(1, 32), lw["w2"],
          lw["b2"].reshape(1, 32), lw["w3"], lw["b3"].reshape(1, 16),
          lw["wl"], lw["bl"].reshape(1, cout))

    mean2, var = pl.pallas_call(
        functools.partial(_head_body, Tt),
        grid=(B,),
        in_specs=[
            pl.BlockSpec((Tt, 8), lambda b: (2 * b + 1, 0)),
            pl.BlockSpec((8, 1), lambda b: (0, 0)),
            pl.BlockSpec((1, 1), lambda b: (0, 0)),
            pl.BlockSpec((8, 1), lambda b: (0, 0)),
            pl.BlockSpec((1, 1), lambda b: (0, 0)),
        ],
        out_specs=[
            pl.BlockSpec((Tt, 1), lambda b: (b, 0)),
            pl.BlockSpec((1, Tt, Tt), lambda b: (b, 0, 0)),
        ],
        out_shape=[
            jax.ShapeDtypeStruct((B * Tt, 1), f32),
            jax.ShapeDtypeStruct((B, Tt, Tt), f32),
        ],
    )(v, params["mean_W"], params["mean_b"].reshape(1, 1),
      params["var_W"], params["var_b"].reshape(1, 1))

    mean = jnp.reshape(mean2, (B, Tt))
    return (mean, var)


# speedup vs baseline: 7.6385x; 1.0173x over previous
import functools

import jax
import jax.numpy as jnp
import numpy as np
from jax import lax
from jax.experimental import pallas as pl
from jax.experimental.pallas import tpu as pltpu
from jax.experimental.pallas import tpu_sc as plsc

_SC_CORES = 2
_SC_SUBCORES = 16
_SC_WORKERS = _SC_CORES * _SC_SUBCORES

_NUM_NBHD = 25
_COEFF = 0.3
_TG = 1024
_C_HI = np.float32(0.1)
_C_LO = np.float32(np.float64(0.1) - np.float64(_C_HI))
_D_HI = np.float32(0.2)
_D_LO = np.float32(np.float64(0.2) - np.float64(_D_HI))

_RT = 256
_RN = 256


def _swish(x):
    return x * jax.nn.sigmoid(x)


def _two_sum(a, b):
    s = a + b
    bb = s - a
    return s, (a - (s - bb)) + (b - bb)


def _feat_body(C, t_row_ref, ctx_ref, vals_ref, preW_ref, preb_ref, sc_ref,
               v0_ref):
    t_row = t_row_ref[:, :]
    ctx = jnp.reshape(ctx_ref[:, :, :], (1, C))
    vals = vals_ref[:, :]
    ls = sc_ref[0, 0]
    os_ = sc_ref[0, 1]
    dd = (t_row - ctx) / ls
    K = os_ * jnp.exp(-0.5 * dd * dd)
    h0 = jnp.sum(K, axis=1, keepdims=True)
    h1 = jnp.dot(K, vals, preferred_element_type=jnp.float32)
    tv = jnp.concatenate([t_row, h0, h1 / (h0 + 1e-8)], axis=1)
    z = jnp.dot(tv, preW_ref[:, :], preferred_element_type=jnp.float32)
    v0_ref[:, :] = jax.nn.sigmoid(z + preb_ref[:, :])


def _knn_body(N, t_row_ref, t_full_ref, pen_ref, idx_ref):
    t_row = t_row_ref[:, :]
    t_full = jnp.reshape(t_full_ref[:, :, :], (1, N))
    pen = jnp.reshape(pen_ref[:, :, :], (1, N))
    d = jnp.abs(t_row - t_full) + pen
    cols = jax.lax.broadcasted_iota(jnp.int32, d.shape, 1)
    base = pl.program_id(0) * N
    picked = []
    for _ in range(_NUM_NBHD):
        m = jnp.min(d, axis=1, keepdims=True)
        i = jnp.min(jnp.where(d == m, cols, N), axis=1, keepdims=True)
        picked.append(i + base)
        d = jnp.where(cols == i, jnp.inf, d)
    idx_ref[:, :] = jnp.concatenate(picked, axis=1)


def _sc_gather(table, idx_flat, D):
    G = idx_flat.shape[0]
    b_per_w = G // _SC_WORKERS
    ch = 400
    n_ch = b_per_w // ch
    mesh = plsc.VectorSubcoreMesh(core_axis_name="c", subcore_axis_name="s")

    @functools.partial(
        pl.kernel, mesh=mesh,
        out_type=jax.ShapeDtypeStruct((G, D), jnp.float32),
        scratch_types=[
            pltpu.VMEM((ch,), jnp.int32),
            pltpu.VMEM((ch,), jnp.int32),
            pltpu.VMEM((ch, D), jnp.float32),
            pltpu.VMEM((ch, D), jnp.float32),
            pltpu.SemaphoreType.DMA,
            pltpu.SemaphoreType.DMA,
        ],
    )
    def k(table_hbm, idx_hbm, out_hbm, idx_v0, idx_v1, rows_v0, rows_v1,
          sem0, sem1):
        idx_bufs = (idx_v0, idx_v1)
        row_bufs = (rows_v0, rows_v1)
        sems = (sem0, sem1)
        wid = lax.axis_index("s") * _SC_CORES + lax.axis_index("c")
        base = wid * b_per_w
        cps = [None, None]
        for ci in range(n_ch):
            b = ci & 1
            if cps[b] is not None:
                cps[b].wait()
                pltpu.sync_copy(row_bufs[b],
                                out_hbm.at[pl.ds(base + (ci - 2) * ch, ch)])
            pltpu.sync_copy(idx_hbm.at[pl.ds(base + ci * ch, ch)],
                            idx_bufs[b])
            cps[b] = pltpu.async_copy(table_hbm.at[idx_bufs[b]],
                                      row_bufs[b], sems[b])
        for ci in range(max(0, n_ch - 2), n_ch):
            b = ci & 1
            cps[b].wait()
            pltpu.sync_copy(row_bufs[b],
                            out_hbm.at[pl.ds(base + ci * ch, ch)])

    return k(table, idx_flat)


def _combine_body(cin, g_ref, t_row_ref, w1_ref, b1_ref, w2_ref, b2_ref,
                  w3_ref, b3_ref, wl_ref, bl_ref, out_ref):
    f32 = jnp.float32
    t_row = t_row_ref[:, :]
    W = cin + 1
    cf = cin * 16
    je = jax.lax.broadcasted_iota(jnp.int32, (cin, cf), 1)
    re = jax.lax.broadcasted_iota(jnp.int32, (cin, cf), 0)
    E1 = (je // 16 == re).astype(f32)
    jf = jax.lax.broadcasted_iota(jnp.int32, (16, cf), 1)
    rf = jax.lax.broadcasted_iota(jnp.int32, (16, cf), 0)
    E2 = (jf % 16 == rf).astype(f32)
    acc = jnp.zeros((_RN, cf), dtype=f32)
    for k in range(_NUM_NBHD):
        gk = g_ref[k, :, :]
        nv = gk[:, :cin]
        nt = gk[:, cin:cin + 1]
        a = t_row - nt
        h = _swish(a * w1_ref[:, :] + b1_ref[:, :])
        h = _swish(jnp.dot(h, w2_ref[:, :], preferred_element_type=f32)
                   + b2_ref[:, :])
        w = (jnp.dot(h, w3_ref[:, :], preferred_element_type=f32)
             + b3_ref[:, :]) * _COEFF
        acc = acc + jnp.dot(nv, E1, preferred_element_type=f32) * \
            jnp.dot(w, E2, preferred_element_type=f32)
    flat = acc / _NUM_NBHD
    z = jnp.dot(flat, wl_ref[:, :], preferred_element_type=f32)
    out_ref[:, :] = jax.nn.relu(z + bl_ref[:, :])


def _head_body(Tt, f_ref, mW_ref, mb_ref, vW_ref, vb_ref, mean_ref, var_ref):
    f = f_ref[:, :]
    mean_ref[:, :] = (jnp.dot(f, mW_ref[:, :], preferred_element_type=jnp.float32)
                      + mb_ref[:, :])
    vd = jax.nn.softplus(
        jnp.dot(f, vW_ref[:, :], preferred_element_type=jnp.float32)
        + vb_ref[:, :])
    r = jax.lax.broadcasted_iota(jnp.int32, (Tt, Tt), 0)
    c = jax.lax.broadcasted_iota(jnp.int32, (Tt, Tt), 1)
    var_ref[0, :, :] = jnp.where(r == c, jnp.maximum(vd, 1e-8),
                                 jnp.float32(1e-8))


def kernel(ctx_coords, ctx_values, tgt_coords, params):
    B, C, _ = ctx_coords.shape
    Tt = tgt_coords.shape[1]
    N = _TG + Tt
    f32 = jnp.float32

    tmp = jnp.concatenate([jnp.reshape(ctx_coords, (-1,)),
                           jnp.reshape(tgt_coords, (-1,))])
    mn = jnp.min(tmp)
    mx = jnp.max(tmp)
    s_lo, e_lo = _two_sum(mn, -_C_HI)
    lower = s_lo + (e_lo - _C_LO)
    s_hi, e_hi = _two_sum(mx, _C_HI)
    upper = s_hi + (e_hi + _C_LO)
    d_s, d_e = _two_sum(mx, -mn)
    p_s, p_e = _two_sum(d_s, _D_HI)
    q = (p_e + d_e) + _D_LO
    a = 64.0 * p_s
    fa = jnp.floor(a)
    num_t_f = fa + jnp.floor((a - fa) + 64.0 * q)
    num_t = jnp.maximum(num_t_f, 1.0).astype(jnp.int32)

    div = jnp.maximum(num_t - 1, 1).astype(f32)
    delta = (upper - lower) / div
    iota = jnp.arange(_TG, dtype=f32)
    t_grid = lower + iota * delta
    t = jnp.concatenate(
        [jnp.broadcast_to(t_grid[None, :, None], (B, _TG, 1)), tgt_coords],
        axis=1)
    t_flat = jnp.reshape(t, (B * N, 1))
    t_wide = jnp.reshape(t, (B, 1, N))

    valid = jnp.concatenate([jnp.arange(_TG) < num_t,
                             jnp.ones((Tt,), dtype=bool)])
    pen = jnp.where(valid, f32(0), f32(jnp.inf)).reshape(1, 1, N)

    ctx_wide = jnp.reshape(ctx_coords, (B, 1, C))
    vals_flat = jnp.reshape(ctx_values, (B * C, 1))
    sc = jnp.stack([jnp.exp(params["log_lengthscale"]),
                    jnp.exp(params["log_outputscale"])]).reshape(1, 2)

    nt_r = N // _RT
    full = lambda b, r: (b, 0)
    wide = lambda b, r: (b, 0, 0)
    cst = lambda b, r: (0, 0)
    cst3 = lambda b, r: (0, 0, 0)
    row_rt = lambda b, r: (b * nt_r + r, 0)

    v0 = pl.pallas_call(
        functools.partial(_feat_body, C),
        grid=(B, nt_r),
        in_specs=[
            pl.BlockSpec((_RT, 1), row_rt),
            pl.BlockSpec((1, 1, C), wide),
            pl.BlockSpec((C, 1), full),
            pl.BlockSpec((3, 8), cst),
            pl.BlockSpec((1, 8), cst),
            pl.BlockSpec((1, 2), cst),
        ],
        out_specs=pl.BlockSpec((_RT, 8), row_rt),
        out_shape=jax.ShapeDtypeStruct((B * N, 8), f32),
    )(t_flat, ctx_wide, vals_flat, params["pre_W"],
      params["pre_b"].reshape(1, 8), sc)

    idx = pl.pallas_call(
        functools.partial(_knn_body, N),
        grid=(B, nt_r),
        in_specs=[
            pl.BlockSpec((_RT, 1), row_rt),
            pl.BlockSpec((1, 1, N), wide),
            pl.BlockSpec((1, 1, N), cst3),
        ],
        out_specs=pl.BlockSpec((_RT, 25), row_rt),
        out_shape=jax.ShapeDtypeStruct((B * N, 25), jnp.int32),
    )(t_flat, t_wide, pen)

    gidx_flat = jnp.reshape(jnp.swapaxes(idx, 0, 1), (-1,))

    v = v0
    M = B * N
    nblk = M // _RN
    row1 = lambda i: (i, 0)
    cst1 = lambda i: (0, 0)
    for lw in params["layers"]:
        cin = v.shape[1]
        cout = lw["wl"].shape[1]
        D = 128
        vt = jnp.concatenate(
            [v, t_flat, jnp.zeros((M, D - cin - 1), f32)], axis=1)
        g = _sc_gather(vt, gidx_flat, D)
        g3 = jnp.reshape(g, (_NUM_NBHD, M, D))
        v = pl.pallas_call(
            functools.partial(_combine_body, cin),
            grid=(nblk,),
            in_specs=[
                pl.BlockSpec((_NUM_NBHD, _RN, D), lambda i: (0, i, 0)),
                pl.BlockSpec((_RN, 1), row1),
                pl.BlockSpec((1, 32), cst1),
                pl.BlockSpec((1, 32), cst1),
                pl.BlockSpec((32, 32), cst1),
                pl.BlockSpec((1, 32), cst1),
                pl.BlockSpec((32, 16), cst1),
                pl.BlockSpec((1, 16), cst1),
                pl.BlockSpec((cin * 16, cout), cst1),
                pl.BlockSpec((1, cout), cst1),
            ],
            out_specs=pl.BlockSpec((_RN, cout), row1),
            out_shape=jax.ShapeDtypeStruct((M, cout), f32),
        )(g3, t_flat, lw["w1"], lw["b1"].reshape(1, 32), lw["w2"],
          lw["b2"].reshape(1, 32), lw["w3"], lw["b3"].reshape(1, 16),
          lw["wl"], lw["bl"].reshape(1, cout))

    mean2, var = pl.pallas_call(
        functools.partial(_head_body, Tt),
        grid=(B,),
        in_specs=[
            pl.BlockSpec((Tt, 8), lambda b: (2 * b + 1, 0)),
            pl.BlockSpec((8, 1), lambda b: (0, 0)),
            pl.BlockSpec((1, 1), lambda b: (0, 0)),
            pl.BlockSpec((8, 1), lambda b: (0, 0)),
            pl.BlockSpec((1, 1), lambda b: (0, 0)),
        ],
        out_specs=[
            pl.BlockSpec((Tt, 1), lambda b: (b, 0)),
            pl.BlockSpec((1, Tt, Tt), lambda b: (b, 0, 0)),
        ],
        out_shape=[
            jax.ShapeDtypeStruct((B * Tt, 1), f32),
            jax.ShapeDtypeStruct((B, Tt, Tt), f32),
        ],
    )(v, params["mean_W"], params["mean_b"].reshape(1, 1),
      params["var_W"], params["var_b"].reshape(1, 1))

    mean = jnp.reshape(mean2, (B, Tt))
    return (mean, var)


# speedup vs baseline: 8.8352x; 1.1567x over previous
import functools

import jax
import jax.numpy as jnp
import numpy as np
from jax import lax
from jax.experimental import pallas as pl
from jax.experimental.pallas import tpu as pltpu
from jax.experimental.pallas import tpu_sc as plsc

_SC_CORES = 2
_SC_SUBCORES = 16
_SC_WORKERS = _SC_CORES * _SC_SUBCORES

_NUM_NBHD = 25
_COEFF = 0.3
_TG = 1024
_C_HI = np.float32(0.1)
_C_LO = np.float32(np.float64(0.1) - np.float64(_C_HI))
_D_HI = np.float32(0.2)
_D_LO = np.float32(np.float64(0.2) - np.float64(_D_HI))

_RT = 256
_RN = 256


def _swish(x):
    return x * jax.nn.sigmoid(x)


def _two_sum(a, b):
    s = a + b
    bb = s - a
    return s, (a - (s - bb)) + (b - bb)


def _feat_body(C, t_row_ref, ctx_ref, vals_ref, preW_ref, preb_ref, sc_ref,
               v0_ref):
    t_row = t_row_ref[:, :]
    ctx = jnp.reshape(ctx_ref[:, :, :], (1, C))
    vals = vals_ref[:, :]
    ls = sc_ref[0, 0]
    os_ = sc_ref[0, 1]
    dd = (t_row - ctx) / ls
    K = os_ * jnp.exp(-0.5 * dd * dd)
    h0 = jnp.sum(K, axis=1, keepdims=True)
    h1 = jnp.dot(K, vals, preferred_element_type=jnp.float32)
    tv = jnp.concatenate([t_row, h0, h1 / (h0 + 1e-8)], axis=1)
    z = jnp.dot(tv, preW_ref[:, :], preferred_element_type=jnp.float32)
    v0_ref[:, :] = jax.nn.sigmoid(z + preb_ref[:, :])


def _knn_body(N, t_row_ref, t_full_ref, pen_ref, idx_ref):
    t_row = t_row_ref[:, :]
    t_full = jnp.reshape(t_full_ref[:, :, :], (1, N))
    pen = jnp.reshape(pen_ref[:, :, :], (1, N))
    d = jnp.abs(t_row - t_full) + pen
    cols = jax.lax.broadcasted_iota(jnp.int32, d.shape, 1)
    base = pl.program_id(0) * N
    picked = []
    for _ in range(_NUM_NBHD):
        m = jnp.min(d, axis=1, keepdims=True)
        i = jnp.min(jnp.where(d == m, cols, N), axis=1, keepdims=True)
        picked.append(i + base)
        d = jnp.where(cols == i, jnp.inf, d)
    idx_ref[:, :] = jnp.concatenate(picked, axis=1)


def _sc_gather(table, idx_flat, D):
    G = idx_flat.shape[0]
    b_per_w = G // _SC_WORKERS
    ch = 400
    n_ch = b_per_w // ch
    mesh = plsc.VectorSubcoreMesh(core_axis_name="c", subcore_axis_name="s")

    @functools.partial(
        pl.kernel, mesh=mesh,
        out_type=jax.ShapeDtypeStruct((G, D), jnp.float32),
        scratch_types=[
            pltpu.VMEM((ch,), jnp.int32),
            pltpu.VMEM((ch,), jnp.int32),
            pltpu.VMEM((ch, D), jnp.float32),
            pltpu.VMEM((ch, D), jnp.float32),
            pltpu.SemaphoreType.DMA,
            pltpu.SemaphoreType.DMA,
        ],
    )
    def k(table_hbm, idx_hbm, out_hbm, idx_v0, idx_v1, rows_v0, rows_v1,
          sem0, sem1):
        idx_bufs = (idx_v0, idx_v1)
        row_bufs = (rows_v0, rows_v1)
        sems = (sem0, sem1)
        wid = lax.axis_index("s") * _SC_CORES + lax.axis_index("c")
        base = wid * b_per_w
        cps = [None, None]
        for ci in range(n_ch):
            b = ci & 1
            if cps[b] is not None:
                cps[b].wait()
                pltpu.sync_copy(row_bufs[b],
                                out_hbm.at[pl.ds(base + (ci - 2) * ch, ch)])
            pltpu.sync_copy(idx_hbm.at[pl.ds(base + ci * ch, ch)],
                            idx_bufs[b])
            cps[b] = pltpu.async_copy(table_hbm.at[idx_bufs[b]],
                                      row_bufs[b], sems[b])
        for ci in range(max(0, n_ch - 2), n_ch):
            b = ci & 1
            cps[b].wait()
            pltpu.sync_copy(row_bufs[b],
                            out_hbm.at[pl.ds(base + ci * ch, ch)])

    return k(table, idx_flat)


def _combine_body(cin, g_ref, t_row_ref, w1_ref, b1_ref, w2_ref, b2_ref,
                  w3_ref, b3_ref, wl_ref, bl_ref, out_ref):
    f32 = jnp.float32
    t_row = t_row_ref[:, :]
    cf = cin * 16
    je = jax.lax.broadcasted_iota(jnp.int32, (cin, cf), 1)
    re = jax.lax.broadcasted_iota(jnp.int32, (cin, cf), 0)
    E1 = (je // 16 == re).astype(f32)
    jf = jax.lax.broadcasted_iota(jnp.int32, (16, cf), 1)
    rf = jax.lax.broadcasted_iota(jnp.int32, (16, cf), 0)
    E2 = (jf % 16 == rf).astype(f32)
    acc = jnp.zeros((_RN, cf), dtype=f32)
    for k in range(_NUM_NBHD):
        gk = g_ref[k, :, :]
        nv = gk[:, :cin]
        nt = gk[:, cin:cin + 1]
        a = t_row - nt
        h = _swish(a * w1_ref[:, :] + b1_ref[:, :])
        h = _swish(jnp.dot(h, w2_ref[:, :], preferred_element_type=f32)
                   + b2_ref[:, :])
        w = (jnp.dot(h, w3_ref[:, :], preferred_element_type=f32)
             + b3_ref[:, :]) * _COEFF
        acc = acc + jnp.dot(nv, E1, preferred_element_type=f32) * \
            jnp.dot(w, E2, preferred_element_type=f32)
    flat = acc / _NUM_NBHD
    z = jnp.dot(flat, wl_ref[:, :], preferred_element_type=f32)
    out_ref[:, :] = jax.nn.relu(z + bl_ref[:, :])


def _head_body(Tt, f_ref, mW_ref, mb_ref, vW_ref, vb_ref, mean_ref, var_ref):
    f = f_ref[:, :]
    mean_ref[:, :] = (jnp.dot(f, mW_ref[:, :], preferred_element_type=jnp.float32)
                      + mb_ref[:, :])
    vd = jax.nn.softplus(
        jnp.dot(f, vW_ref[:, :], preferred_element_type=jnp.float32)
        + vb_ref[:, :])
    r = jax.lax.broadcasted_iota(jnp.int32, (Tt, Tt), 0)
    c = jax.lax.broadcasted_iota(jnp.int32, (Tt, Tt), 1)
    var_ref[0, :, :] = jnp.where(r == c, jnp.maximum(vd, 1e-8),
                                 jnp.float32(1e-8))


def kernel(ctx_coords, ctx_values, tgt_coords, params):
    B, C, _ = ctx_coords.shape
    Tt = tgt_coords.shape[1]
    N = _TG + Tt
    f32 = jnp.float32

    tmp = jnp.concatenate([jnp.reshape(ctx_coords, (-1,)),
                           jnp.reshape(tgt_coords, (-1,))])
    mn = jnp.min(tmp)
    mx = jnp.max(tmp)
    s_lo, e_lo = _two_sum(mn, -_C_HI)
    lower = s_lo + (e_lo - _C_LO)
    s_hi, e_hi = _two_sum(mx, _C_HI)
    upper = s_hi + (e_hi + _C_LO)
    d_s, d_e = _two_sum(mx, -mn)
    p_s, p_e = _two_sum(d_s, _D_HI)
    q = (p_e + d_e) + _D_LO
    a = 64.0 * p_s
    fa = jnp.floor(a)
    num_t_f = fa + jnp.floor((a - fa) + 64.0 * q)
    num_t = jnp.maximum(num_t_f, 1.0).astype(jnp.int32)

    div = jnp.maximum(num_t - 1, 1).astype(f32)
    delta = (upper - lower) / div
    iota = jnp.arange(_TG, dtype=f32)
    t_grid = lower + iota * delta
    t = jnp.concatenate(
        [jnp.broadcast_to(t_grid[None, :, None], (B, _TG, 1)), tgt_coords],
        axis=1)
    t_flat = jnp.reshape(t, (B * N, 1))
    t_wide = jnp.reshape(t, (B, 1, N))

    valid = jnp.concatenate([jnp.arange(_TG) < num_t,
                             jnp.ones((Tt,), dtype=bool)])
    pen = jnp.where(valid, f32(0), f32(jnp.inf)).reshape(1, 1, N)

    ctx_wide = jnp.reshape(ctx_coords, (B, 1, C))
    vals_flat = jnp.reshape(ctx_values, (B * C, 1))
    sc = jnp.stack([jnp.exp(params["log_lengthscale"]),
                    jnp.exp(params["log_outputscale"])]).reshape(1, 2)

    nt_r = N // _RT
    full = lambda b, r: (b, 0)
    wide = lambda b, r: (b, 0, 0)
    cst = lambda b, r: (0, 0)
    cst3 = lambda b, r: (0, 0, 0)
    row_rt = lambda b, r: (b * nt_r + r, 0)

    v0 = pl.pallas_call(
        functools.partial(_feat_body, C),
        grid=(B, nt_r),
        in_specs=[
            pl.BlockSpec((_RT, 1), row_rt),
            pl.BlockSpec((1, 1, C), wide),
            pl.BlockSpec((C, 1), full),
            pl.BlockSpec((3, 8), cst),
            pl.BlockSpec((1, 8), cst),
            pl.BlockSpec((1, 2), cst),
        ],
        out_specs=pl.BlockSpec((_RT, 8), row_rt),
        out_shape=jax.ShapeDtypeStruct((B * N, 8), f32),
    )(t_flat, ctx_wide, vals_flat, params["pre_W"],
      params["pre_b"].reshape(1, 8), sc)

    idx = pl.pallas_call(
        functools.partial(_knn_body, N),
        grid=(B, nt_r),
        in_specs=[
            pl.BlockSpec((_RT, 1), row_rt),
            pl.BlockSpec((1, 1, N), wide),
            pl.BlockSpec((1, 1, N), cst3),
        ],
        out_specs=pl.BlockSpec((_RT, 25), row_rt),
        out_shape=jax.ShapeDtypeStruct((B * N, 25), jnp.int32),
    )(t_flat, t_wide, pen)

    gidx_flat = jnp.reshape(jnp.swapaxes(idx, 0, 1), (-1,))
    M4 = B * Tt
    idx4 = jnp.reshape(jnp.reshape(idx, (B, N, 25))[:, _TG:, :], (M4, 25))
    gidx4_flat = jnp.reshape(jnp.swapaxes(idx4, 0, 1), (-1,))
    t_tgt = jnp.reshape(tgt_coords, (M4, 1))

    v = v0
    M = B * N
    row1 = lambda i: (i, 0)
    cst1 = lambda i: (0, 0)
    for li, lw in enumerate(params["layers"]):
        cin = v.shape[1]
        cout = lw["wl"].shape[1]
        last = li == len(params["layers"]) - 1
        gi = gidx4_flat if last else gidx_flat
        t_in = t_tgt if last else t_flat
        Mo = M4 if last else M
        nblk = Mo // _RN
        D = 128
        vt = jnp.concatenate(
            [v, t_flat, jnp.zeros((M, D - cin - 1), f32)], axis=1)
        g = _sc_gather(vt, gi, D)
        g3 = jnp.reshape(g, (_NUM_NBHD, Mo, D))
        v = pl.pallas_call(
            functools.partial(_combine_body, cin),
            grid=(nblk,),
            in_specs=[
                pl.BlockSpec((_NUM_NBHD, _RN, D), lambda i: (0, i, 0)),
                pl.BlockSpec((_RN, 1), row1),
                pl.BlockSpec((1, 32), cst1),
                pl.BlockSpec((1, 32), cst1),
                pl.BlockSpec((32, 32), cst1),
                pl.BlockSpec((1, 32), cst1),
                pl.BlockSpec((32, 16), cst1),
                pl.BlockSpec((1, 16), cst1),
                pl.BlockSpec((cin * 16, cout), cst1),
                pl.BlockSpec((1, cout), cst1),
            ],
            out_specs=pl.BlockSpec((_RN, cout), row1),
            out_shape=jax.ShapeDtypeStruct((Mo, cout), f32),
        )(g3, t_in, lw["w1"], lw["b1"].reshape(1, 32), lw["w2"],
          lw["b2"].reshape(1, 32), lw["w3"], lw["b3"].reshape(1, 16),
          lw["wl"], lw["bl"].reshape(1, cout))

    mean2, var = pl.pallas_call(
        functools.partial(_head_body, Tt),
        grid=(B,),
        in_specs=[
            pl.BlockSpec((Tt, 8), lambda b: (b, 0)),
            pl.BlockSpec((8, 1), lambda b: (0, 0)),
            pl.BlockSpec((1, 1), lambda b: (0, 0)),
            pl.BlockSpec((8, 1), lambda b: (0, 0)),
            pl.BlockSpec((1, 1), lambda b: (0, 0)),
        ],
        out_specs=[
            pl.BlockSpec((Tt, 1), lambda b: (b, 0)),
            pl.BlockSpec((1, Tt, Tt), lambda b: (b, 0, 0)),
        ],
        out_shape=[
            jax.ShapeDtypeStruct((B * Tt, 1), f32),
            jax.ShapeDtypeStruct((B, Tt, Tt), f32),
        ],
    )(v, params["mean_W"], params["mean_b"].reshape(1, 1),
      params["var_W"], params["var_b"].reshape(1, 1))

    mean = jnp.reshape(mean2, (B, Tt))
    return (mean, var)
